# baseline (speedup 1.0000x reference)
import jax
import jax.numpy as jnp
import numpy as np
from jax.experimental import pallas as pl
from jax.experimental.pallas import tpu as pltpu

N_BINS = 15
_BOUNDS = np.linspace(0.0, 1.0, N_BINS + 1, dtype=np.float32)
_CB = 8
_SUB = 8
_SPLIT = 504


def _update(x_ref, m_ref, b_ref, base0):
    m = m_ref[...]
    b = b_ref[...]
    for j in range(_CB // _SUB):
        sub = x_ref[_SUB * j:_SUB * (j + 1), :]
        upd = sub > m
        m = jnp.where(upd, sub, m)
        b = jnp.where(upd, base0 + j * _SUB, b)
    m_ref[...] = m
    b_ref[...] = b


def _ece_kernel(x1_ref, x2_ref, conf_ref, lab_ref, bounds_ref, out_ref,
                m1_ref, b1_ref, m2_ref, b2_ref):
    i = pl.program_id(0)
    nb = pl.num_programs(0)

    @pl.when(i == 0)
    def _init():
        m1_ref[...] = jnp.full_like(m1_ref, -jnp.inf)
        b1_ref[...] = jnp.zeros_like(b1_ref)
        m2_ref[...] = jnp.full_like(m2_ref, -jnp.inf)
        b2_ref[...] = jnp.zeros_like(b2_ref)

    _update(x1_ref, m1_ref, b1_ref, i * _CB)
    i2 = jnp.minimum(i, (1000 - _SPLIT) // _CB - 1)
    _update(x2_ref, m2_ref, b2_ref, _SPLIT + i2 * _CB)

    @pl.when(i == nb - 1)
    def _finish():
        m1 = m1_ref[...]
        m2 = m2_ref[...]
        take1 = m1 >= m2
        mm = jnp.where(take1, m1, m2)
        bb = jnp.where(take1, b1_ref[...], b2_ref[...])
        idx = bb + jax.lax.broadcasted_iota(jnp.int32, mm.shape, 0)
        gmax = jnp.max(mm, axis=0, keepdims=True)
        ji = jnp.where(mm == gmax, idx, jnp.int32(1 << 30))
        fmi = jnp.min(ji, axis=0, keepdims=True)
        acc = (fmi == lab_ref[...]).astype(jnp.float32)

        conf = conf_ref[...]
        n = conf.shape[1]
        conf_b = jnp.broadcast_to(conf, (_SUB, n))
        acc_b = jnp.broadcast_to(acc, (_SUB, n))
        ece = jnp.zeros((1, 1), jnp.float32)
        for g in range(2):
            lob = bounds_ref[_SUB * g:_SUB * (g + 1), 0:1]
            hib = bounds_ref[_SUB * g:_SUB * (g + 1), 1:2]
            mask = ((conf_b > lob) & (conf_b <= hib)).astype(jnp.float32)
            cnt = jnp.sum(mask, axis=1, keepdims=True)
            sumc = jnp.sum(mask * conf_b, axis=1, keepdims=True)
            suma = jnp.sum(mask * acc_b, axis=1, keepdims=True)
            safe = jnp.where(cnt > 0, cnt, 1.0)
            contrib = jnp.where(
                cnt > 0,
                jnp.abs(sumc / safe - suma / safe) * (cnt / n),
                0.0,
            )
            ece += jnp.sum(contrib).reshape(1, 1)
        out_ref[...] = ece


def kernel(softmaxes, confidences, labels):
    n, c = softmaxes.shape
    xt = softmaxes.T
    nb = _SPLIT // _CB
    nb2 = (c - _SPLIT) // _CB
    conf2 = confidences.reshape(1, n)
    lab2 = labels.astype(jnp.int32).reshape(1, n)
    bnp = np.full((16, 2), 2.0, dtype=np.float32)
    bnp[:N_BINS, 0] = _BOUNDS[:-1]
    bnp[:N_BINS, 1] = _BOUNDS[1:]
    bounds = jnp.asarray(bnp)
    split_blk = _SPLIT // _CB

    out = pl.pallas_call(
        _ece_kernel,
        grid=(nb,),
        in_specs=[
            pl.BlockSpec((_CB, n), lambda i: (i, 0)),
            pl.BlockSpec(
                (_CB, n),
                lambda i: (split_blk + jnp.minimum(i, nb2 - 1), 0),
            ),
            pl.BlockSpec((1, n), lambda i: (0, 0)),
            pl.BlockSpec((1, n), lambda i: (0, 0)),
            pl.BlockSpec((16, 2), lambda i: (0, 0)),
        ],
        out_specs=pl.BlockSpec((1, 1), lambda i: (0, 0)),
        out_shape=jax.ShapeDtypeStruct((1, 1), jnp.float32),
        scratch_shapes=[
            pltpu.VMEM((_SUB, n), jnp.float32),
            pltpu.VMEM((_SUB, n), jnp.int32),
            pltpu.VMEM((_SUB, n), jnp.float32),
            pltpu.VMEM((_SUB, n), jnp.int32),
        ],
    )(xt, xt, conf2, lab2, bounds)
    return out.reshape(1)


# speedup vs baseline: 1.3130x; 1.3130x over previous
import jax
import jax.numpy as jnp
import numpy as np
from jax.experimental import pallas as pl
from jax.experimental.pallas import tpu as pltpu

N_BINS = 15
_BOUNDS = np.linspace(0.0, 1.0, N_BINS + 1, dtype=np.float32)
_CB1 = 32
_CB2 = 8
_SUB = 8
_SPLIT = 800


def _update(x_ref, m_ref, b_ref, base0, cb):
    m = m_ref[...]
    b = b_ref[...]
    for j in range(cb // _SUB):
        sub = x_ref[_SUB * j:_SUB * (j + 1), :]
        upd = sub > m
        m = jnp.where(upd, sub, m)
        b = jnp.where(upd, base0 + j * _SUB, b)
    m_ref[...] = m
    b_ref[...] = b


def _ece_kernel(x1_ref, x2_ref, conf_ref, lab_ref, bounds_ref, out_ref,
                m1_ref, b1_ref, m2_ref, b2_ref):
    i = pl.program_id(0)
    nb = pl.num_programs(0)

    @pl.when(i == 0)
    def _init():
        m1_ref[...] = jnp.full_like(m1_ref, -jnp.inf)
        b1_ref[...] = jnp.zeros_like(b1_ref)
        m2_ref[...] = jnp.full_like(m2_ref, -jnp.inf)
        b2_ref[...] = jnp.zeros_like(b2_ref)

    _update(x1_ref, m1_ref, b1_ref, i * _CB1, _CB1)
    _update(x2_ref, m2_ref, b2_ref, _SPLIT + i * _CB2, _CB2)

    @pl.when(i == nb - 1)
    def _finish():
        m1 = m1_ref[...]
        m2 = m2_ref[...]
        take1 = m1 >= m2
        mm = jnp.where(take1, m1, m2)
        bb = jnp.where(take1, b1_ref[...], b2_ref[...])
        idx = bb + jax.lax.broadcasted_iota(jnp.int32, mm.shape, 0)
        gmax = jnp.max(mm, axis=0, keepdims=True)
        ji = jnp.where(mm == gmax, idx, jnp.int32(1 << 30))
        fmi = jnp.min(ji, axis=0, keepdims=True)
        acc = (fmi == lab_ref[...]).astype(jnp.float32)

        conf = conf_ref[...]
        n = conf.shape[1]
        conf_b = jnp.broadcast_to(conf, (_SUB, n))
        acc_b = jnp.broadcast_to(acc, (_SUB, n))
        ece = jnp.zeros((1, 1), jnp.float32)
        for g in range(2):
            lob = bounds_ref[_SUB * g:_SUB * (g + 1), 0:1]
            hib = bounds_ref[_SUB * g:_SUB * (g + 1), 1:2]
            mask = ((conf_b > lob) & (conf_b <= hib)).astype(jnp.float32)
            cnt = jnp.sum(mask, axis=1, keepdims=True)
            sumc = jnp.sum(mask * conf_b, axis=1, keepdims=True)
            suma = jnp.sum(mask * acc_b, axis=1, keepdims=True)
            safe = jnp.where(cnt > 0, cnt, 1.0)
            contrib = jnp.where(
                cnt > 0,
                jnp.abs(sumc / safe - suma / safe) * (cnt / n),
                0.0,
            )
            ece += jnp.sum(contrib).reshape(1, 1)
        out_ref[...] = ece


def kernel(softmaxes, confidences, labels):
    n, c = softmaxes.shape
    xt = softmaxes.T
    nb = _SPLIT // _CB1
    conf2 = confidences.reshape(1, n)
    lab2 = labels.astype(jnp.int32).reshape(1, n)
    bnp = np.full((16, 2), 2.0, dtype=np.float32)
    bnp[:N_BINS, 0] = _BOUNDS[:-1]
    bnp[:N_BINS, 1] = _BOUNDS[1:]
    bounds = jnp.asarray(bnp)
    split_blk = _SPLIT // _CB2

    out = pl.pallas_call(
        _ece_kernel,
        grid=(nb,),
        in_specs=[
            pl.BlockSpec((_CB1, n), lambda i: (i, 0)),
            pl.BlockSpec((_CB2, n), lambda i: (split_blk + i, 0)),
            pl.BlockSpec((1, n), lambda i: (0, 0)),
            pl.BlockSpec((1, n), lambda i: (0, 0)),
            pl.BlockSpec((16, 2), lambda i: (0, 0)),
        ],
        out_specs=pl.BlockSpec((1, 1), lambda i: (0, 0)),
        out_shape=jax.ShapeDtypeStruct((1, 1), jnp.float32),
        scratch_shapes=[
            pltpu.VMEM((_SUB, n), jnp.float32),
            pltpu.VMEM((_SUB, n), jnp.int32),
            pltpu.VMEM((_SUB, n), jnp.float32),
            pltpu.VMEM((_SUB, n), jnp.int32),
        ],
    )(xt, xt, conf2, lab2, bounds)
    return out.reshape(1)
